# baseline (speedup 1.0000x reference)
import functools

import jax
import jax.numpy as jnp
from jax import lax
from jax.experimental import pallas as pl
from jax.experimental.pallas import tpu as pltpu
from jax.experimental.pallas import tpu_sc as plsc

_L = 16
_NBUF = 5
_TCB = 2048


_H = 512000


def _repack_table(table_t):
    d, v = table_t.shape
    grid = _H // _TCB
    off = _H // _TCB
    last = (v + _TCB - 1) // _TCB - 1

    def body(x1_ref, x2_ref, o_ref):
        o_ref[:, 0:d] = x1_ref[...].T
        o_ref[:, d:2 * d] = x2_ref[...].T

    return pl.pallas_call(
        body,
        grid=(grid,),
        in_specs=[
            pl.BlockSpec((d, _TCB), lambda i: (0, i)),
            pl.BlockSpec((d, _TCB), lambda i: (0, jnp.minimum(i + off, last))),
        ],
        out_specs=pl.BlockSpec((_TCB, 2 * d), lambda i: (i, 0)),
        out_shape=jax.ShapeDtypeStruct((_H, 2 * d), jnp.float32),
    )(table_t, table_t)


def _build(num_workers, per_w, ch, u_rows, d, r, with_lora):
    n_ch = per_w // ch
    assert n_ch % _NBUF == 0
    mesh = plsc.VectorSubcoreMesh(core_axis_name="c", subcore_axis_name="s")
    d2 = 2 * d

    scratch = (
        [pltpu.VMEM((n_ch, ch), jnp.int32)]
        + [pltpu.VMEM((n_ch, ch), jnp.int32)]
        + [pltpu.VMEM((ch, d2), jnp.float32) for _ in range(_NBUF)]
        + ([pltpu.VMEM((ch, r), jnp.float32) for _ in range(_NBUF)] if with_lora else [])
        + ([pltpu.VMEM((r, d), jnp.float32)] if with_lora else [])
        + [pltpu.SemaphoreType.DMA for _ in range((3 if with_lora else 2) * _NBUF)]
    )

    @functools.partial(
        pl.kernel,
        mesh=mesh,
        compiler_params=pltpu.CompilerParams(use_tc_tiling_on_sc=False, needs_layout_passes=False),
        out_type=jax.ShapeDtypeStruct((num_workers * per_w // 2, d2), jnp.float32),
        scratch_types=scratch,
    )
    def k(u_tab, rid, sel, *rest):
        if with_lora:
            a_tab, b_tab, out = rest[0], rest[1], rest[2]
            rest = rest[3:]
        else:
            out = rest[0]
            rest = rest[1:]
        idx_v = rest[0]
        sel_v = rest[1]
        rows = list(rest[2:2 + _NBUF])
        rest = rest[2 + _NBUF:]
        if with_lora:
            avs = list(rest[:_NBUF])
            b_v = rest[_NBUF]
            rest = rest[_NBUF + 1:]
        sem_t = list(rest[:_NBUF])
        sem_o = list(rest[_NBUF:2 * _NBUF])
        if with_lora:
            sem_a = list(rest[2 * _NBUF:])

        nc = 2
        wid = lax.axis_index("s") * nc + lax.axis_index("c")
        base2 = wid * (per_w // 2)
        pltpu.sync_copy(rid.at[wid], idx_v)
        pltpu.sync_copy(sel.at[wid], sel_v)
        if with_lora:
            pltpu.sync_copy(b_tab, b_v)

        def gather_start(ci, b):
            pltpu.make_async_copy(u_tab.at[idx_v.at[ci]], rows[b], sem_t[b]).start()
            if with_lora:
                pltpu.make_async_copy(a_tab.at[idx_v.at[ci]], avs[b], sem_a[b]).start()

        def gather_wait(ci, b):
            pltpu.make_async_copy(u_tab.at[idx_v.at[ci]], rows[b], sem_t[b]).wait()
            if with_lora:
                pltpu.make_async_copy(a_tab.at[idx_v.at[ci]], avs[b], sem_a[b]).wait()

        def out_start(g, b):
            pltpu.make_async_copy(
                rows[b].at[pl.ds(0, ch // 2)],
                out.at[pl.ds(base2 + g * (ch // 2), ch // 2)],
                sem_o[b]).start()

        def out_wait(g, b):
            pltpu.make_async_copy(
                rows[b].at[pl.ds(0, ch // 2)],
                out.at[pl.ds(base2 + g * (ch // 2), ch // 2)],
                sem_o[b]).wait()

        def compute(ci, b):
            def grp_body(g, _):
                i0 = g * _L
                r16 = i0 + lax.iota(jnp.int32, _L)
                sv = sel_v[ci, pl.ds(i0, _L)]
                s64 = sv * d
                wrow = r16 // 2
                wcol = (r16 % 2) * d
                for j in range(d):
                    vals = plsc.load_gather(rows[b], [r16, s64 + j])
                    plsc.store_scatter(rows[b], [wrow, wcol + j], vals)
                return 0

            lax.fori_loop(0, ch // _L, grp_body, 0, unroll=False)

            if with_lora:
                def row_body(i, _):
                    av = avs[b][i, pl.ds(0, r)]
                    for c in range(d // _L):
                        off = (i % 2) * d + c * _L
                        acc = rows[b][i // 2, pl.ds(off, _L)]
                        for rr in range(r):
                            acc = acc + av[rr] * b_v[rr, pl.ds(c * _L, _L)]
                        rows[b][i // 2, pl.ds(off, _L)] = acc
                    return 0

                lax.fori_loop(0, ch, row_body, 0, unroll=False)

        for b in range(_NBUF - 1):
            gather_start(b, b)

        for g in range(_NBUF):
            b = g
            if g >= 1:
                out_wait(g - 1, g - 1)
            gather_start(g + _NBUF - 1, (g + _NBUF - 1) % _NBUF)
            gather_wait(g, b)
            compute(g, b)
            out_start(g, b)

        def outer_body(it, _):
            o = it * _NBUF
            for b in range(_NBUF):
                g = o + b
                out_wait(g - 1, (b - 1) % _NBUF)

                @pl.when(g + _NBUF - 1 < n_ch)
                def _():
                    gather_start(g + _NBUF - 1, (b + _NBUF - 1) % _NBUF)

                gather_wait(g, b)
                compute(g, b)
                out_start(g, b)
            return 0

        lax.fori_loop(1, n_ch // _NBUF, outer_body, 0, unroll=False)

        out_wait(n_ch - 1, (n_ch - 1) % _NBUF)

    return k


def kernel(embedding_weight, input_ids, lora_A, lora_B):
    v, d = embedding_weight.shape
    r = lora_A.shape[1]
    b, s = input_ids.shape
    n = b * s
    nw = 32
    ch = 128
    per_w = n // nw

    ids = input_ids.reshape(nw, per_w // ch, ch).astype(jnp.int32)
    rid = ids % _H
    sel = ids // _H

    u_tab = _repack_table(embedding_weight.T)
    u_rows = u_tab.shape[0]

    k_full = _build(nw, per_w, ch, u_rows, d, r, with_lora=True)
    k_plain = _build(nw, per_w, ch, u_rows, d, r, with_lora=False)

    def full_branch(u, rid_, sel_, a_tab, b_tab):
        return k_full(u, rid_, sel_, a_tab, b_tab)

    def plain_branch(u, rid_, sel_, a_tab, b_tab):
        return k_plain(u, rid_, sel_)

    has_lora = jnp.any(lora_B != 0.0)
    out = lax.cond(has_lora, full_branch, plain_branch,
                   u_tab, rid, sel, lora_A, lora_B)
    return out.reshape(b, s, d)


# speedup vs baseline: 1.0010x; 1.0010x over previous
import functools

import jax
import jax.numpy as jnp
from jax import lax
from jax.experimental import pallas as pl
from jax.experimental.pallas import tpu as pltpu
from jax.experimental.pallas import tpu_sc as plsc

_L = 16
_NBUF = 5
_TCB = 2048


_H = 512000


def _repack_table(table_t):
    d, v = table_t.shape
    grid = _H // _TCB
    off = _H // _TCB
    last = (v + _TCB - 1) // _TCB - 1

    def body(x1_ref, x2_ref, o_ref):
        eye = (lax.broadcasted_iota(jnp.int32, (d, d), 0)
               == lax.broadcasted_iota(jnp.int32, (d, d), 1)).astype(jnp.float32)
        dn = (((0,), (0,)), ((), ()))
        o_ref[:, 0:d] = lax.dot_general(
            x1_ref[...], eye, dn, preferred_element_type=jnp.float32)
        o_ref[:, d:2 * d] = lax.dot_general(
            x2_ref[...], eye, dn, preferred_element_type=jnp.float32)

    return pl.pallas_call(
        body,
        grid=(grid,),
        in_specs=[
            pl.BlockSpec((d, _TCB), lambda i: (0, i)),
            pl.BlockSpec((d, _TCB), lambda i: (0, jnp.minimum(i + off, last))),
        ],
        out_specs=pl.BlockSpec((_TCB, 2 * d), lambda i: (i, 0)),
        out_shape=jax.ShapeDtypeStruct((_H, 2 * d), jnp.float32),
    )(table_t, table_t)


def _build(num_workers, per_w, ch, u_rows, d, r, with_lora):
    n_ch = per_w // ch
    assert n_ch % _NBUF == 0
    mesh = plsc.VectorSubcoreMesh(core_axis_name="c", subcore_axis_name="s")
    d2 = 2 * d

    scratch = (
        [pltpu.VMEM((n_ch, ch), jnp.int32)]
        + [pltpu.VMEM((n_ch, ch), jnp.int32)]
        + [pltpu.VMEM((ch, d2), jnp.float32) for _ in range(_NBUF)]
        + [pltpu.VMEM((ch // 2, d2), jnp.float32) for _ in range(_NBUF)]
        + ([pltpu.VMEM((ch, r), jnp.float32) for _ in range(_NBUF)] if with_lora else [])
        + ([pltpu.VMEM((r, d), jnp.float32)] if with_lora else [])
        + [pltpu.SemaphoreType.DMA for _ in range((3 if with_lora else 2) * _NBUF)]
    )

    @functools.partial(
        pl.kernel,
        mesh=mesh,
        compiler_params=pltpu.CompilerParams(use_tc_tiling_on_sc=False, needs_layout_passes=False),
        out_type=jax.ShapeDtypeStruct((num_workers * per_w // 2, d2), jnp.float32),
        scratch_types=scratch,
    )
    def k(u_tab, rid, sel, *rest):
        if with_lora:
            a_tab, b_tab, out = rest[0], rest[1], rest[2]
            rest = rest[3:]
        else:
            out = rest[0]
            rest = rest[1:]
        idx_v = rest[0]
        sel_v = rest[1]
        rows = list(rest[2:2 + _NBUF])
        stage = list(rest[2 + _NBUF:2 + 2 * _NBUF])
        rest = rest[2 + 2 * _NBUF:]
        if with_lora:
            avs = list(rest[:_NBUF])
            b_v = rest[_NBUF]
            rest = rest[_NBUF + 1:]
        sem_t = list(rest[:_NBUF])
        sem_o = list(rest[_NBUF:2 * _NBUF])
        if with_lora:
            sem_a = list(rest[2 * _NBUF:])

        nc = 2
        wid = lax.axis_index("s") * nc + lax.axis_index("c")
        base2 = wid * (per_w // 2)
        pltpu.sync_copy(rid.at[wid], idx_v)
        pltpu.sync_copy(sel.at[wid], sel_v)
        if with_lora:
            pltpu.sync_copy(b_tab, b_v)

        def gather_start(ci, b):
            pltpu.make_async_copy(u_tab.at[idx_v.at[ci]], rows[b], sem_t[b]).start()
            if with_lora:
                pltpu.make_async_copy(a_tab.at[idx_v.at[ci]], avs[b], sem_a[b]).start()

        def gather_wait(ci, b):
            pltpu.make_async_copy(u_tab.at[idx_v.at[ci]], rows[b], sem_t[b]).wait()
            if with_lora:
                pltpu.make_async_copy(a_tab.at[idx_v.at[ci]], avs[b], sem_a[b]).wait()

        def out_start(g, b):
            pltpu.make_async_copy(
                stage[b],
                out.at[pl.ds(base2 + g * (ch // 2), ch // 2)],
                sem_o[b]).start()

        def out_wait(g, b):
            pltpu.make_async_copy(
                stage[b],
                out.at[pl.ds(base2 + g * (ch // 2), ch // 2)],
                sem_o[b]).wait()

        def compute(ci, b):
            def grp_body(g, _):
                i0 = g * _L
                r16 = i0 + lax.iota(jnp.int32, _L)
                sv = sel_v[ci, pl.ds(i0, _L)]
                s64 = sv * d
                wrow = r16 // 2
                wcol = (r16 % 2) * d
                for j in range(d):
                    vals = plsc.load_gather(rows[b], [r16, s64 + j])
                    plsc.store_scatter(stage[b], [wrow, wcol + j], vals)
                return 0

            lax.fori_loop(0, ch // _L, grp_body, 0, unroll=False)

            if with_lora:
                def row_body(i, _):
                    av = avs[b][i, pl.ds(0, r)]
                    for c in range(d // _L):
                        off = (i % 2) * d + c * _L
                        acc = stage[b][i // 2, pl.ds(off, _L)]
                        for rr in range(r):
                            acc = acc + av[rr] * b_v[rr, pl.ds(c * _L, _L)]
                        stage[b][i // 2, pl.ds(off, _L)] = acc
                    return 0

                lax.fori_loop(0, ch, row_body, 0, unroll=False)

        for b in range(_NBUF - 1):
            gather_start(b, b)

        for g in range(_NBUF):
            b = g
            if g >= 1:
                out_wait(g - 1, g - 1)
            gather_start(g + _NBUF - 1, (g + _NBUF - 1) % _NBUF)
            gather_wait(g, b)
            compute(g, b)
            out_start(g, b)

        def outer_body(it, _):
            o = it * _NBUF
            for b in range(_NBUF):
                g = o + b
                out_wait(g - 1, (b - 1) % _NBUF)

                @pl.when(g + _NBUF - 1 < n_ch)
                def _():
                    gather_start(g + _NBUF - 1, (b + _NBUF - 1) % _NBUF)

                gather_wait(g, b)
                compute(g, b)
                out_start(g, b)
            return 0

        lax.fori_loop(1, n_ch // _NBUF, outer_body, 0, unroll=False)

        out_wait(n_ch - 1, (n_ch - 1) % _NBUF)

    return k


def kernel(embedding_weight, input_ids, lora_A, lora_B):
    v, d = embedding_weight.shape
    r = lora_A.shape[1]
    b, s = input_ids.shape
    n = b * s
    nw = 32
    ch = 80
    per_w = n // nw

    ids = input_ids.reshape(nw, per_w // ch, ch).astype(jnp.int32)
    rid = ids % _H
    sel = ids // _H

    u_tab = _repack_table(embedding_weight.T)
    u_rows = u_tab.shape[0]

    k_full = _build(nw, per_w, ch, u_rows, d, r, with_lora=True)
    k_plain = _build(nw, per_w, ch, u_rows, d, r, with_lora=False)

    def full_branch(u, rid_, sel_, a_tab, b_tab):
        return k_full(u, rid_, sel_, a_tab, b_tab)

    def plain_branch(u, rid_, sel_, a_tab, b_tab):
        return k_plain(u, rid_, sel_)

    has_lora = jnp.any(lora_B != 0.0)
    out = lax.cond(has_lora, full_branch, plain_branch,
                   u_tab, rid, sel, lora_A, lora_B)
    return out.reshape(b, s, d)


# speedup vs baseline: 1.0829x; 1.0818x over previous
import functools

import jax
import jax.numpy as jnp
from jax import lax
from jax.experimental import pallas as pl
from jax.experimental.pallas import tpu as pltpu
from jax.experimental.pallas import tpu_sc as plsc

_L = 16
_NBUF = 5


def _build(num_workers, per_w, ch, v, d, r, with_lora):
    n_ch = per_w // ch
    assert n_ch % _NBUF == 0
    mesh = plsc.VectorSubcoreMesh(core_axis_name="c", subcore_axis_name="s")

    scratch = (
        [pltpu.VMEM((n_ch, ch), jnp.int32)]
        + [pltpu.VMEM((ch, d), jnp.float32) for _ in range(_NBUF)]
        + ([pltpu.VMEM((ch, r), jnp.float32) for _ in range(_NBUF)] if with_lora else [])
        + ([pltpu.VMEM((r, d), jnp.float32)] if with_lora else [])
        + [pltpu.SemaphoreType.DMA for _ in range((3 if with_lora else 2) * _NBUF)]
    )

    @functools.partial(
        pl.kernel,
        mesh=mesh,
        compiler_params=pltpu.CompilerParams(use_tc_tiling_on_sc=False),
        out_type=jax.ShapeDtypeStruct((num_workers * per_w, d), jnp.float32),
        scratch_types=scratch,
    )
    def k(table, ids, *rest):
        if with_lora:
            a_tab, b_tab, out = rest[0], rest[1], rest[2]
            rest = rest[3:]
        else:
            out = rest[0]
            rest = rest[1:]
        idx_v = rest[0]
        rest = rest[1:]
        rows = list(rest[:_NBUF])
        rest = rest[_NBUF:]
        if with_lora:
            avs = list(rest[:_NBUF])
            b_v = rest[_NBUF]
            rest = rest[_NBUF + 1:]
        sem_t = list(rest[:_NBUF])
        sem_o = list(rest[_NBUF:2 * _NBUF])
        if with_lora:
            sem_a = list(rest[2 * _NBUF:])

        nc = 2
        wid = lax.axis_index("s") * nc + lax.axis_index("c")
        base = wid * per_w
        pltpu.sync_copy(ids.at[wid], idx_v)
        if with_lora:
            pltpu.sync_copy(b_tab, b_v)

        def gather_start(ci, b):
            pltpu.make_async_copy(table.at[idx_v.at[ci]], rows[b], sem_t[b]).start()
            if with_lora:
                pltpu.make_async_copy(a_tab.at[idx_v.at[ci]], avs[b], sem_a[b]).start()

        def gather_wait(ci, b):
            pltpu.make_async_copy(table.at[idx_v.at[ci]], rows[b], sem_t[b]).wait()
            if with_lora:
                pltpu.make_async_copy(a_tab.at[idx_v.at[ci]], avs[b], sem_a[b]).wait()

        def out_start(g, b):
            pltpu.make_async_copy(
                rows[b], out.at[pl.ds(base + g * ch, ch)], sem_o[b]).start()

        def out_wait(g, b):
            pltpu.make_async_copy(
                rows[b], out.at[pl.ds(base + g * ch, ch)], sem_o[b]).wait()

        def compute(b):
            if not with_lora:
                return

            def row_body(i, _):
                av = avs[b][i, pl.ds(0, r)]
                for c in range(d // _L):
                    acc = rows[b][i, pl.ds(c * _L, _L)]
                    for rr in range(r):
                        acc = acc + av[rr] * b_v[rr, pl.ds(c * _L, _L)]
                    rows[b][i, pl.ds(c * _L, _L)] = acc
                return 0

            lax.fori_loop(0, ch, row_body, 0, unroll=False)

        for b in range(_NBUF - 1):
            gather_start(b, b)

        for g in range(_NBUF):
            b = g
            if g >= 1:
                out_wait(g - 1, g - 1)
            gather_start(g + _NBUF - 1, (g + _NBUF - 1) % _NBUF)
            gather_wait(g, b)
            compute(b)
            out_start(g, b)

        def outer_body(it, _):
            o = it * _NBUF
            for b in range(_NBUF):
                g = o + b
                out_wait(g - 1, (b - 1) % _NBUF)

                @pl.when(g + _NBUF - 1 < n_ch)
                def _():
                    gather_start(g + _NBUF - 1, (b + _NBUF - 1) % _NBUF)

                gather_wait(g, b)
                compute(b)
                out_start(g, b)
            return 0

        lax.fori_loop(1, n_ch // _NBUF, outer_body, 0, unroll=False)

        out_wait(n_ch - 1, (n_ch - 1) % _NBUF)

    return k


def kernel(embedding_weight, input_ids, lora_A, lora_B):
    v, d = embedding_weight.shape
    r = lora_A.shape[1]
    b, s = input_ids.shape
    n = b * s
    nw = 32
    ch = 128
    per_w = n // nw
    ids = input_ids.reshape(nw, per_w // ch, ch).astype(jnp.int32)

    k_full = _build(nw, per_w, ch, v, d, r, with_lora=True)
    k_plain = _build(nw, per_w, ch, v, d, r, with_lora=False)

    def full_branch(table, idx, a_tab, b_tab):
        return k_full(table, idx, a_tab, b_tab)

    def plain_branch(table, idx, a_tab, b_tab):
        return k_plain(table, idx)

    has_lora = jnp.any(lora_B != 0.0)
    out = lax.cond(has_lora, full_branch, plain_branch,
                   embedding_weight, ids, lora_A, lora_B)
    return out.reshape(b, s, d)
